# baseline (speedup 1.0000x reference)
import functools

import jax
import jax.numpy as jnp
from jax import lax
from jax.experimental import pallas as pl
from jax.experimental.pallas import tpu as pltpu
from jax.experimental.pallas import tpu_sc as plsc

N = 10000
E = 320000
D_IN = 128
HID = 128
NC, NS = 2, 16
NW = NC * NS
NB = 2
NV = NW * NB
R = 157
NPAD = NV * R
W = 128
EBLK = 256
FLUSH = 4096
CAP = E + 2 * FLUSH
ES = E // NW
KP = 2000
FCAP = 10528
FSTGS = 528
FL = 432
TH = 432
PKS = 16384
ACC = (R + 1) * W
NEG = float("-inf")
POS = float("inf")

_mesh = plsc.VectorSubcoreMesh(core_axis_name="c", subcore_axis_name="s")
_sc_params = pltpu.CompilerParams(needs_layout_passes=False)


def _wid():
    return lax.axis_index("s") * NC + lax.axis_index("c")


def _part_body(edges, fragpk, fraghist, s_scan, d_scan, stg, ptrs, fcnt,
               histp):
    wid = _wid()
    iota = lax.iota(jnp.int32, 16)
    lane0 = iota == 0
    one0 = jnp.where(lane0, 1, 0)
    ones16 = jnp.full((16,), 1, jnp.int32)
    tmask = jnp.full((16,), True)
    zero_i = jnp.zeros((16,), jnp.int32)

    def zh(i, _):
        histp[pl.ds(i * 16, 16)] = zero_i
        return 0

    lax.fori_loop(0, (64 * 176) // 16, zh, 0)
    for i in range(80 // 16):
        ptrs[pl.ds(i * 16, 16)] = zero_i
        fcnt[pl.ds(i * 16, 16)] = zero_i

    ebase = wid * ES

    def chunk(k, _):
        pltpu.sync_copy(edges.at[pl.ds(ebase + k * KP, KP)], s_scan)
        pltpu.sync_copy(edges.at[pl.ds(E + ebase + k * KP, KP)], d_scan)

        def unit(q, _):
            for u in range(5):
                g = q * 5 + u
                sv = s_scan[pl.ds(g * 16, 16)]
                dv = d_scan[pl.ds(g * 16, 16)]
                binv = lax.shift_right_logical(dv * 106862, 24)
                ldstv = dv - binv * R
                plsc.addupdate_scatter(histp, [binv * 176 + ldstv], ones16,
                                       mask=tmask)
                pkv = sv + ldstv * PKS
                for lane in range(16):
                    b = binv[lane]
                    pk = pkv[lane]
                    bsl = pl.ds(b, 16)
                    pv = ptrs[bsl]
                    p = pv[0]
                    ptrs[bsl] = pv + one0
                    addr = b * FSTGS + p
                    plsc.store_scatter(stg,
                                       [jnp.full((16,), addr, jnp.int32)],
                                       jnp.full((16,), pk, jnp.int32),
                                       mask=lane0)
            for qq in range(4):
                pv = ptrs[pl.ds(qq * 16, 16)]
                mx = plsc.cummax(pv)[15]

                @pl.when(mx >= TH)
                def _(pv=pv, qq=qq):
                    for lane in range(16):
                        b = qq * 16 + lane
                        p = pv[lane]

                        @pl.when(p >= TH)
                        def _(b=b, p=p):
                            fsl = pl.ds(b, 16)
                            fv = pl.multiple_of(fcnt[fsl][0], 16)
                            off = pl.multiple_of(
                                (wid * 64 + b) * FCAP + fv, 8)
                            pltpu.sync_copy(
                                stg.at[pl.ds(b * FSTGS, FL)],
                                fragpk.at[pl.ds(off, FL)])
                            for j in range(6):
                                stg[pl.ds(b * FSTGS + j * 16, 16)] = (
                                    stg[pl.ds(b * FSTGS + FL + j * 16, 16)])
                            fcnt[fsl] = fcnt[fsl] + jnp.where(lane0, FL, 0)
                            ptrs[fsl] = ptrs[fsl] - jnp.where(lane0, FL, 0)
            return 0

        lax.fori_loop(0, KP // 80, unit, 0)
        return 0

    lax.fori_loop(0, ES // KP, chunk, 0)

    def fin(b, _):
        psl = pl.ds(b, 16)
        p = ptrs[psl][0]
        sent = R * PKS + b * R + iota
        plsc.store_scatter(stg, [b * FSTGS + p + iota], sent, mask=tmask)
        p16 = ((p + 15) // 16) * 16
        hsl = pl.ds(b * 176 + R, 16)
        histp[hsl] = histp[hsl] + jnp.where(lane0, p16 - p, 0)
        fv = pl.multiple_of(fcnt[psl][0], 16)
        off = pl.multiple_of((wid * 64 + b) * FCAP + fv, 8)
        pltpu.sync_copy(stg.at[pl.ds(pl.multiple_of(b * FSTGS, 8), 512)],
                        fragpk.at[pl.ds(off, 512)])
        pltpu.sync_copy(
            histp.at[pl.ds(pl.multiple_of(b * 176, 8), 176)],
            fraghist.at[pl.ds(pl.multiple_of((b * 32 + wid) * 176, 8), 176)])
        return 0

    lax.fori_loop(0, 64, fin, 0)


_part_edges = functools.partial(
    pl.kernel,
    out_type=[
        jax.ShapeDtypeStruct((NW * 64 * FCAP,), jnp.int32),
        jax.ShapeDtypeStruct((NV * 32 * 176,), jnp.int32),
    ],
    mesh=_mesh,
    scratch_types=[
        pltpu.VMEM((KP,), jnp.int32),
        pltpu.VMEM((KP,), jnp.int32),
        pltpu.VMEM((64 * FSTGS,), jnp.int32),
        pltpu.VMEM((80,), jnp.int32),
        pltpu.VMEM((80,), jnp.int32),
        pltpu.VMEM((64 * 176,), jnp.int32),
    ],
    compiler_params=_sc_params,
)(_part_body)


CAPS = 49152
SBLK = 4096


def _sort_body(fragpk, fraghist, osrcl, ooffl, odeg, ocnt,
               pkblk, osrc, wo, histv, offv, hblk, fcv, degf, cbuf):
    wid = _wid()
    iota = lax.iota(jnp.int32, 16)
    zero_i = jnp.zeros((16,), jnp.int32)
    one0 = jnp.where(iota == 0, 1, 0)
    lane0 = iota == 0

    for b in range(NB):
        v = wid * NB + b

        pltpu.sync_copy(fraghist.at[pl.ds(v * 32 * 176, 32 * 176)], hblk)
        for i in range(176 // 16):
            histv[pl.ds(i * 16, 16)] = zero_i

        def sumt(t, cnt):
            s = zero_i
            for i in range(176 // 16):
                hv = hblk[pl.ds(t * 176 + i * 16, 16)]
                sl = pl.ds(i * 16, 16)
                histv[sl] = histv[sl] + hv
                s = s + hv
            fc = plsc.cumsum(s)[15]
            wsl = pl.ds(t, 16)
            fcv[wsl] = jnp.where(lane0, fc, fcv[wsl])
            return cnt + fc

        cnt = lax.fori_loop(0, 32, sumt, jnp.int32(0))
        cnt_out = ((cnt + EBLK - 1) // EBLK) * EBLK

        for i in range(160 // 16):
            degf[pl.ds(i * 16, 16)] = histv[pl.ds(i * 16, 16)].astype(
                jnp.float32)
        pltpu.sync_copy(degf, odeg.at[pl.ds(v * 160, 160)])

        carry = jnp.int32(0)
        for i in range(176 // 16):
            hv = histv[pl.ds(i * 16, 16)]
            ps = plsc.cumsum(hv)
            offv[pl.ds(i * 16, 16)] = ps - hv + carry
            carry = carry + ps[15]
        for i in range(176 // 16):
            idx = i * 16 + iota
            sl = pl.ds(i * 16, 16)
            offv[sl] = jnp.where(idx >= R + 1, cnt_out, offv[sl])

        pltpu.sync_copy(offv.at[pl.ds(0, 160)], ooffl.at[pl.ds(v * 160, 160)])
        cbuf[...] = jnp.full((16,), cnt_out, jnp.int32)
        pltpu.sync_copy(cbuf, ocnt.at[pl.ds(v * 16, 16)])

        npass = (cnt_out + CAPS - 1) // CAPS

        def ppass(w, _):
            w0 = pl.multiple_of(w * CAPS, 8)
            for i in range(176 // 16):
                wo[pl.ds(i * 16, 16)] = offv[pl.ds(i * 16, 16)]

            def pfrag(t, _):
                fc = fcv[pl.ds(t, 16)][0]
                nfb = (fc + SBLK - 1) // SBLK

                def pblk(bi, _):
                    boff = pl.multiple_of(bi * SBLK, 8)
                    fb = pl.multiple_of((t * 64 + v) * FCAP, 8)
                    pltpu.sync_copy(fragpk.at[pl.ds(fb + boff, SBLK)], pkblk)
                    nin = jnp.minimum(fc - bi * SBLK, SBLK)

                    def pgrp(g, _):
                        pk = pkblk[pl.ds(g * 16, 16)]
                        dvec = lax.shift_right_logical(pk, 14)
                        svec = pk & (PKS - 1)
                        for lane in range(16):
                            d = dvec[lane]
                            s = svec[lane]
                            dsl = pl.ds(d, 16)
                            ov = wo[dsl]
                            p = ov[0]
                            wo[dsl] = ov + one0
                            pw = p - w0
                            inwin = (pw >= 0) & (pw < CAPS)
                            mk = lane0 & jnp.full((16,), inwin)
                            pv = jnp.full((16,), pw, jnp.int32)
                            plsc.store_scatter(osrc, [pv],
                                               jnp.full((16,), s, jnp.int32),
                                               mask=mk)
                        return 0

                    lax.fori_loop(0, nin // 16, pgrp, 0)
                    return 0

                lax.fori_loop(0, nfb, pblk, 0)
                return 0

            lax.fori_loop(0, 32, pfrag, 0)

            for j in range(EBLK // 16):
                pos = cnt + j * 16 + iota
                pw = pos - w0
                msk = (pos < cnt_out) & (pw >= 0) & (pw < CAPS)
                plsc.store_scatter(osrc, [pw], v * R + iota, mask=msk)

            nw = (jnp.minimum(cnt_out - w0, CAPS) + SBLK - 1) // SBLK

            def wblk(bi, _):
                boff = pl.multiple_of(bi * SBLK, 8)
                pltpu.sync_copy(osrc.at[pl.ds(boff, SBLK)],
                                osrcl.at[pl.ds(v * CAP + w0 + boff, SBLK)])
                return 0

            lax.fori_loop(0, nw, wblk, 0)
            return 0

        lax.fori_loop(0, npass, ppass, 0)


_sort_bins = functools.partial(
    pl.kernel,
    out_type=[
        jax.ShapeDtypeStruct((NV * CAP,), jnp.int32),
        jax.ShapeDtypeStruct((NV * 160,), jnp.int32),
        jax.ShapeDtypeStruct((NV * 160,), jnp.float32),
        jax.ShapeDtypeStruct((NV * 16,), jnp.int32),
    ],
    mesh=_mesh,
    scratch_types=[
        pltpu.VMEM((SBLK,), jnp.int32),
        pltpu.VMEM((CAPS,), jnp.int32),
        pltpu.VMEM((176,), jnp.int32),
        pltpu.VMEM((176,), jnp.int32),
        pltpu.VMEM((176,), jnp.int32),
        pltpu.VMEM((32 * 176,), jnp.int32),
        pltpu.VMEM((48,), jnp.int32),
        pltpu.VMEM((160,), jnp.float32),
        pltpu.VMEM((16,), jnp.int32),
    ],
    compiler_params=_sc_params,
)(_sort_body)


def _agg_body(ncw, c, tbl, srcl, ooffl, cntl, *refs):
    (osum, omax, omin, srcb0, srcb1, idx0, idx1, gbuf0, gbuf1,
     accs, accm, accn, offb, cntb, sem0, sem1) = refs
    wid = _wid()

    zero = jnp.zeros((16,), jnp.float32)
    negs = zero + NEG
    poss = zero + POS
    NJ = W // 16
    bufs = ((srcb0, idx0, gbuf0, sem0), (srcb1, idx1, gbuf1, sem1))

    for b in range(NB):
        v = wid * NB + b

        def initr(i, _):
            sl = pl.ds(i * 16, 16)
            accs[sl] = zero
            accm[sl] = negs
            accn[sl] = poss
            return 0

        lax.fori_loop(0, ACC // 16, initr, 0)

        pltpu.sync_copy(cntl.at[pl.ds(v * 16, 16)], cntb)
        pltpu.sync_copy(ooffl.at[pl.ds(v * 160, 160)], offb.at[pl.ds(0, 160)])
        cnt = cntb[pl.ds(0, 16)][0]
        nblk = cnt // EBLK

        def issue(bi, p):
            srcb, idx2, gbuf, sem = bufs[p]
            boff = pl.multiple_of(bi * EBLK, 8)
            pltpu.sync_copy(srcl.at[pl.ds(v * CAP + boff, EBLK)], srcb)
            for i in range(EBLK // 16):
                vv = srcb[pl.ds(i * 16, 16)] * ncw + c
                idx2[i // 8, pl.ds((i % 8) * 16, 16)] = vv
            for j in range(EBLK // 128):
                pltpu.async_copy(tbl.at[idx2.at[j]],
                                 gbuf.at[pl.ds(j * 128, 128)], sem)

        def wait_g(p):
            srcb, idx2, gbuf, sem = bufs[p]
            for j in range(EBLK // 128):
                pltpu.make_async_copy(tbl.at[idx2.at[j]],
                                      gbuf.at[pl.ds(j * 128, 128)],
                                      sem).wait()

        def compute(bi, p, carry):
            gbuf = bufs[p][2]
            e0 = bi * EBLK

            def seg_cond(st):
                return st[1] < EBLK

            def seg_body(st):
                r = st[0]
                pos = st[1]
                regs = list(st[2:])
                rend = offb[pl.ds(r + 1, 16)][0] - e0
                send = jnp.minimum(rend, EBLK)

                def acc_e(el, regs2):
                    regs2 = list(regs2)
                    for j in range(NJ):
                        rr = gbuf[el, pl.ds(j * 16, 16)]
                        regs2[j] = regs2[j] + rr
                        regs2[NJ + j] = jnp.maximum(regs2[NJ + j], rr)
                        regs2[2 * NJ + j] = jnp.minimum(regs2[2 * NJ + j], rr)
                    return tuple(regs2)

                regs = list(lax.fori_loop(pos, send, acc_e, tuple(regs)))
                fin = rend <= EBLK

                def flush(args, rr=r):
                    for j in range(NJ):
                        sl = pl.ds(rr * W + j * 16, 16)
                        accs[sl] = accs[sl] + args[j]
                        accm[sl] = jnp.maximum(accm[sl], args[NJ + j])
                        accn[sl] = jnp.minimum(accn[sl], args[2 * NJ + j])
                    return ([zero] * NJ) + ([negs] * NJ) + ([poss] * NJ)

                regs = lax.cond(fin, flush, lambda a: list(a), tuple(regs))
                r = jnp.where(fin, r + 1, r)
                return (r, send, *regs)

            st = lax.while_loop(seg_cond, seg_body,
                                (carry[0], jnp.int32(0), *carry[1:]))
            return (st[0], *st[2:])

        issue(0, 0)

        def pair(i, carry):
            b0 = 2 * i
            b1 = 2 * i + 1
            wait_g(0)

            @pl.when(b1 < nblk)
            def _():
                issue(b1, 1)

            carry = compute(b0, 0, carry)

            def second(cc):
                @pl.when(b1 + 1 < nblk)
                def _():
                    issue(b1 + 1, 0)

                wait_g(1)
                return compute(b1, 1, cc)

            return lax.cond(b1 < nblk, second, lambda cc: cc, carry)

        init = (jnp.int32(0),) + tuple([zero] * NJ + [negs] * NJ + [poss] * NJ)
        lax.fori_loop(0, (nblk + 1) // 2, pair, init)

        pltpu.sync_copy(accs.at[pl.ds(0, R * W)],
                        osum.at[pl.ds(v * R * W, R * W)])
        pltpu.sync_copy(accm.at[pl.ds(0, R * W)],
                        omax.at[pl.ds(v * R * W, R * W)])
        pltpu.sync_copy(accn.at[pl.ds(0, R * W)],
                        omin.at[pl.ds(v * R * W, R * W)])


def _make_agg(ncw, c):
    outs = [jax.ShapeDtypeStruct((NPAD * W,), jnp.float32)] * 3
    scratch = [
        pltpu.VMEM((EBLK,), jnp.int32),
        pltpu.VMEM((EBLK,), jnp.int32),
        pltpu.VMEM((EBLK // 128, 128), jnp.int32),
        pltpu.VMEM((EBLK // 128, 128), jnp.int32),
        pltpu.VMEM((EBLK, W), jnp.float32),
        pltpu.VMEM((EBLK, W), jnp.float32),
        pltpu.VMEM((ACC,), jnp.float32),
        pltpu.VMEM((ACC,), jnp.float32),
        pltpu.VMEM((ACC,), jnp.float32),
        pltpu.VMEM((176,), jnp.int32),
        pltpu.VMEM((16,), jnp.int32),
        pltpu.SemaphoreType.DMA,
        pltpu.SemaphoreType.DMA,
    ]
    return pl.kernel(
        functools.partial(_agg_body, ncw, c),
        out_type=outs,
        mesh=_mesh,
        scratch_types=scratch,
        compiler_params=_sc_params,
    )


BR = 2000


def _c1_body(h_ref, sm_ref, mx_ref, mn_ref, dg_ref,
             wl0, wr0, wl1, wr1, wl2, wr2, bl0, bl1, bl2,
             opre, ostat):
    i = pl.program_id(0)
    deg = dg_ref[...]
    degc = jnp.maximum(deg, 1.0)
    emp = deg <= 0.0
    h = h_ref[...]
    mean = sm_ref[...] / degc
    mxv = jnp.where(emp, 0.0, mx_ref[...])
    mnv = jnp.where(emp, 0.0, mn_ref[...])
    parts = []
    for agg, Wl, bl, Wr in ((mean, wl0, bl0, wr0),
                            (mxv, wl1, bl1, wr1),
                            (mnv, wl2, bl2, wr2)):
        parts.append(
            jnp.dot(agg, Wl[...], preferred_element_type=jnp.float32)
            + bl[...]
            + jnp.dot(h, Wr[...], preferred_element_type=jnp.float32))
    pre = jnp.concatenate(parts, axis=1)
    opre[...] = pre

    @pl.when(i == 0)
    def _():
        ostat[...] = jnp.zeros_like(ostat)

    s0 = jnp.sum(pre, axis=0)[None, :]
    s1 = jnp.sum(pre * pre, axis=0)[None, :]
    pad = jnp.zeros((6, pre.shape[1]), jnp.float32)
    ostat[...] = ostat[...] + jnp.concatenate([s0, s1, pad], axis=0)


def _make_c1(K):
    grid = N // BR
    rb = lambda i: (i, 0)
    cb = lambda i: (0, 0)
    return pl.pallas_call(
        _c1_body,
        grid=(grid,),
        in_specs=[
            pl.BlockSpec((BR, K), rb),
            pl.BlockSpec((BR, K), rb),
            pl.BlockSpec((BR, K), rb),
            pl.BlockSpec((BR, K), rb),
            pl.BlockSpec((BR, 1), rb),
        ] + [pl.BlockSpec((K, HID), cb)] * 6 + [pl.BlockSpec((1, HID), cb)] * 3,
        out_specs=[
            pl.BlockSpec((BR, 3 * HID), rb),
            pl.BlockSpec((8, 3 * HID), cb),
        ],
        out_shape=[
            jax.ShapeDtypeStruct((N, 3 * HID), jnp.float32),
            jax.ShapeDtypeStruct((8, 3 * HID), jnp.float32),
        ],
    )


def _c2_body(final, pre_ref, stat_ref, g_ref, b_ref, *rest):
    if final:
        cw_ref, cb_ref, out_ref = rest
    else:
        (out_ref,) = rest
    stat = stat_ref[...]
    mu = stat[0:1, :] / jnp.float32(N)
    var = stat[1:2, :] / jnp.float32(N) - mu * mu
    inv = lax.rsqrt(var + 1e-5)
    h = (pre_ref[...] - mu) * (inv * g_ref[...]) + b_ref[...]
    h = jnp.maximum(h, 0.0)
    if final:
        out_ref[...] = (jnp.dot(h, cw_ref[...],
                                preferred_element_type=jnp.float32)
                        + cb_ref[...])
    else:
        out_ref[...] = h


def _make_c2(final):
    grid = N // BR
    rb = lambda i: (i, 0)
    cb = lambda i: (0, 0)
    K = 3 * HID
    in_specs = [
        pl.BlockSpec((BR, K), rb),
        pl.BlockSpec((8, K), cb),
        pl.BlockSpec((1, K), cb),
        pl.BlockSpec((1, K), cb),
    ]
    if final:
        in_specs += [pl.BlockSpec((K, HID), cb), pl.BlockSpec((1, HID), cb)]
        out_w = HID
    else:
        out_w = K
    return pl.pallas_call(
        functools.partial(_c2_body, final),
        grid=(grid,),
        in_specs=in_specs,
        out_specs=pl.BlockSpec((BR, out_w), rb),
        out_shape=jax.ShapeDtypeStruct((N, out_w), jnp.float32),
    )


def _layer_aggregate(tbl2d, ncw, srcl, ldstl, cnt):
    sums, maxs, mins = [], [], []
    for c in range(ncw):
        s, m, n = _make_agg(ncw, c)(tbl2d, srcl, ldstl, cnt)
        sums.append(s.reshape(NPAD, W))
        maxs.append(m.reshape(NPAD, W))
        mins.append(n.reshape(NPAD, W))
    sm = jnp.concatenate(sums, axis=1)[:N]
    mx = jnp.concatenate(maxs, axis=1)[:N]
    mn = jnp.concatenate(mins, axis=1)[:N]
    return sm, mx, mn


def kernel(x, edge_index,
           Wl_0_0, bl_0_0, Wr_0_0,
           Wl_0_1, bl_0_1, Wr_0_1,
           Wl_0_2, bl_0_2, Wr_0_2,
           bn_g_0, bn_b_0,
           Wl_1_0, bl_1_0, Wr_1_0,
           Wl_1_1, bl_1_1, Wr_1_1,
           Wl_1_2, bl_1_2, Wr_1_2,
           bn_g_1, bn_b_1,
           clf_W, clf_b):
    fragpk, fraghist = _part_edges(edge_index.reshape(2 * E))
    srcl, ldstl, deg, cnt = _sort_bins(fragpk, fraghist)

    sm0, mx0, mn0 = _layer_aggregate(x, D_IN // W, srcl, ldstl, cnt)
    degv = deg.reshape(NV, 160)[:, :R].reshape(NPAD, 1)[:N]
    c1 = _make_c1(D_IN)
    pre0, stat0 = c1(x, sm0, mx0, mn0, degv,
                     Wl_0_0, Wr_0_0, Wl_0_1, Wr_0_1, Wl_0_2, Wr_0_2,
                     bl_0_0.reshape(1, HID), bl_0_1.reshape(1, HID),
                     bl_0_2.reshape(1, HID))
    h1 = _make_c2(False)(pre0, stat0, bn_g_0.reshape(1, -1),
                         bn_b_0.reshape(1, -1))

    tbl1 = h1.reshape(N * (3 * HID // W), W)
    sm1, mx1, mn1 = _layer_aggregate(tbl1, 3 * HID // W, srcl, ldstl, cnt)
    c1b = _make_c1(3 * HID)
    pre1, stat1 = c1b(h1, sm1, mx1, mn1, degv,
                      Wl_1_0, Wr_1_0, Wl_1_1, Wr_1_1, Wl_1_2, Wr_1_2,
                      bl_1_0.reshape(1, HID), bl_1_1.reshape(1, HID),
                      bl_1_2.reshape(1, HID))
    clf_Wp = jnp.pad(clf_W, ((0, 0), (0, HID - clf_W.shape[1])))
    clf_bp = jnp.pad(clf_b, (0, HID - clf_b.shape[0])).reshape(1, HID)
    logits = _make_c2(True)(pre1, stat1, bn_g_1.reshape(1, -1),
                            bn_b_1.reshape(1, -1), clf_Wp, clf_bp)
    return logits[:, :clf_W.shape[1]]


# speedup vs baseline: 1.4224x; 1.4224x over previous
import functools

import jax
import jax.numpy as jnp
from jax import lax
from jax.experimental import pallas as pl
from jax.experimental.pallas import tpu as pltpu
from jax.experimental.pallas import tpu_sc as plsc

N = 10000
E = 320000
D_IN = 128
HID = 128
NC, NS = 2, 16
NW = NC * NS
NB = 2
NV = NW * NB
R = 157
NPAD = NV * R
W = 128
EBLK = 256
FLUSH = 4096
CAP = E + 2 * FLUSH
ES = E // NW
KP = 2000
FCAP = 10528
FSTGS = 528
FL = 432
TH = 432
PKS = 16384
ACC = (R + 1) * W
NEG = float("-inf")
POS = float("inf")

_mesh = plsc.VectorSubcoreMesh(core_axis_name="c", subcore_axis_name="s")
_sc_params = pltpu.CompilerParams(needs_layout_passes=False)


def _wid():
    return lax.axis_index("s") * NC + lax.axis_index("c")


def _part_body(edges, fragpk, fraghist, s_scan, d_scan, stg, ptrs, fcnt,
               histp, shb):
    wid = _wid()
    iota = lax.iota(jnp.int32, 16)
    lane0 = iota == 0
    ones16 = jnp.full((16,), 1, jnp.int32)
    tmask = jnp.full((16,), True)
    zero_i = jnp.zeros((16,), jnp.int32)

    def zh(i, _):
        histp[pl.ds(i * 16, 16)] = zero_i
        return 0

    lax.fori_loop(0, (64 * 176) // 16, zh, 0)
    for i in range(80 // 16):
        ptrs[pl.ds(i * 16, 16)] = zero_i
        fcnt[pl.ds(i * 16, 16)] = zero_i
    shb[pl.ds(0, 16)] = jnp.where(lane0, -1, 0)
    shb[pl.ds(16, 16)] = jnp.full((16,), -2, jnp.int32)

    ebase = wid * ES

    def chunk(k, _):
        pltpu.sync_copy(edges.at[pl.ds(ebase + k * KP, KP)], s_scan)
        pltpu.sync_copy(edges.at[pl.ds(E + ebase + k * KP, KP)], d_scan)

        def unit(q, _):
            for u in range(5):
                g = q * 5 + u
                sv = s_scan[pl.ds(g * 16, 16)]
                dv = d_scan[pl.ds(g * 16, 16)]
                binv = lax.shift_right_logical(dv * 106862, 24)
                ldstv = dv - binv * R
                plsc.addupdate_scatter(histp, [binv * 176 + ldstv], ones16,
                                       mask=tmask)
                pkv = binv * (1 << 22) + ldstv * PKS + sv
                spk, _unused = plsc.sort_key_val(pkv, pkv)
                sbin = lax.shift_right_logical(spk, 22)
                slow = spk & ((1 << 22) - 1)
                shb[pl.ds(1, 16)] = sbin
                prev = shb[pl.ds(0, 16)]
                nxt = shb[pl.ds(2, 16)]
                mstart = sbin != prev
                mend = sbin != nxt
                rank = iota - plsc.cummax(jnp.where(mstart, iota, 0))
                pbase = plsc.load_gather(ptrs, [sbin])
                addr = sbin * FSTGS + pbase + rank
                plsc.store_scatter(stg, [addr], slow, mask=tmask)
                plsc.addupdate_scatter(ptrs, [sbin], rank + 1, mask=mend)
            for qq in range(4):
                pv = ptrs[pl.ds(qq * 16, 16)]
                mx = plsc.cummax(pv)[15]

                @pl.when(mx >= TH)
                def _(pv=pv, qq=qq):
                    for lane in range(16):
                        b = qq * 16 + lane
                        p = pv[lane]

                        @pl.when(p >= TH)
                        def _(b=b, p=p):
                            fsl = pl.ds(b, 16)
                            fv = pl.multiple_of(fcnt[fsl][0], 16)
                            off = pl.multiple_of(
                                (wid * 64 + b) * FCAP + fv, 8)
                            pltpu.sync_copy(
                                stg.at[pl.ds(b * FSTGS, FL)],
                                fragpk.at[pl.ds(off, FL)])
                            for j in range(6):
                                stg[pl.ds(b * FSTGS + j * 16, 16)] = (
                                    stg[pl.ds(b * FSTGS + FL + j * 16, 16)])
                            fcnt[fsl] = fcnt[fsl] + jnp.where(lane0, FL, 0)
                            ptrs[fsl] = ptrs[fsl] - jnp.where(lane0, FL, 0)
            return 0

        lax.fori_loop(0, KP // 80, unit, 0)
        return 0

    lax.fori_loop(0, ES // KP, chunk, 0)

    def fin(b, _):
        psl = pl.ds(b, 16)
        p = ptrs[psl][0]
        sent = R * PKS + b * R + iota
        plsc.store_scatter(stg, [b * FSTGS + p + iota], sent, mask=tmask)
        p16 = ((p + 15) // 16) * 16
        hsl = pl.ds(b * 176 + R, 16)
        histp[hsl] = histp[hsl] + jnp.where(lane0, p16 - p, 0)
        fv = pl.multiple_of(fcnt[psl][0], 16)
        off = pl.multiple_of((wid * 64 + b) * FCAP + fv, 8)
        pltpu.sync_copy(stg.at[pl.ds(pl.multiple_of(b * FSTGS, 8), 512)],
                        fragpk.at[pl.ds(off, 512)])
        pltpu.sync_copy(
            histp.at[pl.ds(pl.multiple_of(b * 176, 8), 176)],
            fraghist.at[pl.ds(pl.multiple_of((b * 32 + wid) * 176, 8), 176)])
        return 0

    lax.fori_loop(0, 64, fin, 0)


_part_edges = functools.partial(
    pl.kernel,
    out_type=[
        jax.ShapeDtypeStruct((NW * 64 * FCAP,), jnp.int32),
        jax.ShapeDtypeStruct((NV * 32 * 176,), jnp.int32),
    ],
    mesh=_mesh,
    scratch_types=[
        pltpu.VMEM((KP,), jnp.int32),
        pltpu.VMEM((KP,), jnp.int32),
        pltpu.VMEM((64 * FSTGS,), jnp.int32),
        pltpu.VMEM((80,), jnp.int32),
        pltpu.VMEM((80,), jnp.int32),
        pltpu.VMEM((64 * 176,), jnp.int32),
        pltpu.VMEM((48,), jnp.int32),
    ],
    compiler_params=_sc_params,
)(_part_body)


CAPS = 49152
SBLK = 4096


def _sort_body(fragpk, fraghist, osrcl, ooffl, odeg, ocnt,
               pkblk, osrc, wo, histv, offv, hblk, fcv, degf, cbuf, shb):
    wid = _wid()
    iota = lax.iota(jnp.int32, 16)
    zero_i = jnp.zeros((16,), jnp.int32)
    lane0 = iota == 0
    shb[pl.ds(0, 16)] = jnp.where(lane0, -1, 0)
    shb[pl.ds(16, 16)] = jnp.full((16,), -2, jnp.int32)

    for b in range(NB):
        v = wid * NB + b

        pltpu.sync_copy(fraghist.at[pl.ds(v * 32 * 176, 32 * 176)], hblk)
        for i in range(176 // 16):
            histv[pl.ds(i * 16, 16)] = zero_i

        def sumt(t, cnt):
            s = zero_i
            for i in range(176 // 16):
                hv = hblk[pl.ds(t * 176 + i * 16, 16)]
                sl = pl.ds(i * 16, 16)
                histv[sl] = histv[sl] + hv
                s = s + hv
            fc = plsc.cumsum(s)[15]
            wsl = pl.ds(t, 16)
            fcv[wsl] = jnp.where(lane0, fc, fcv[wsl])
            return cnt + fc

        cnt = lax.fori_loop(0, 32, sumt, jnp.int32(0))
        cnt_out = ((cnt + EBLK - 1) // EBLK) * EBLK

        for i in range(160 // 16):
            degf[pl.ds(i * 16, 16)] = histv[pl.ds(i * 16, 16)].astype(
                jnp.float32)
        pltpu.sync_copy(degf, odeg.at[pl.ds(v * 160, 160)])

        carry = jnp.int32(0)
        for i in range(176 // 16):
            hv = histv[pl.ds(i * 16, 16)]
            ps = plsc.cumsum(hv)
            offv[pl.ds(i * 16, 16)] = ps - hv + carry
            carry = carry + ps[15]
        for i in range(176 // 16):
            idx = i * 16 + iota
            sl = pl.ds(i * 16, 16)
            offv[sl] = jnp.where(idx >= R + 1, cnt_out, offv[sl])

        pltpu.sync_copy(offv.at[pl.ds(0, 160)], ooffl.at[pl.ds(v * 160, 160)])
        cbuf[...] = jnp.full((16,), cnt_out, jnp.int32)
        pltpu.sync_copy(cbuf, ocnt.at[pl.ds(v * 16, 16)])

        npass = (cnt_out + CAPS - 1) // CAPS

        def ppass(w, _):
            w0 = pl.multiple_of(w * CAPS, 8)
            for i in range(176 // 16):
                wo[pl.ds(i * 16, 16)] = offv[pl.ds(i * 16, 16)]

            def pfrag(t, _):
                fc = fcv[pl.ds(t, 16)][0]
                nfb = (fc + SBLK - 1) // SBLK

                def pblk(bi, _):
                    boff = pl.multiple_of(bi * SBLK, 8)
                    fb = pl.multiple_of((t * 64 + v) * FCAP, 8)
                    pltpu.sync_copy(fragpk.at[pl.ds(fb + boff, SBLK)], pkblk)
                    nin = jnp.minimum(fc - bi * SBLK, SBLK)

                    def pgrp(g, _):
                        pk = pkblk[pl.ds(g * 16, 16)]
                        spk, _unused = plsc.sort_key_val(pk, pk)
                        sd = lax.shift_right_logical(spk, 14)
                        ss = spk & (PKS - 1)
                        shb[pl.ds(1, 16)] = sd
                        prev = shb[pl.ds(0, 16)]
                        nxt = shb[pl.ds(2, 16)]
                        mstart = sd != prev
                        mend = sd != nxt
                        rank = iota - plsc.cummax(
                            jnp.where(mstart, iota, 0))
                        obase = plsc.load_gather(wo, [sd])
                        pw = obase + rank - w0
                        inwin = (pw >= 0) & (pw < CAPS)
                        plsc.store_scatter(osrc, [pw], ss, mask=inwin)
                        plsc.addupdate_scatter(wo, [sd], rank + 1, mask=mend)
                        return 0

                    lax.fori_loop(0, nin // 16, pgrp, 0)
                    return 0

                lax.fori_loop(0, nfb, pblk, 0)
                return 0

            lax.fori_loop(0, 32, pfrag, 0)

            for j in range(EBLK // 16):
                pos = cnt + j * 16 + iota
                pw = pos - w0
                msk = (pos < cnt_out) & (pw >= 0) & (pw < CAPS)
                plsc.store_scatter(osrc, [pw], v * R + iota, mask=msk)

            nw = (jnp.minimum(cnt_out - w0, CAPS) + SBLK - 1) // SBLK

            def wblk(bi, _):
                boff = pl.multiple_of(bi * SBLK, 8)
                pltpu.sync_copy(osrc.at[pl.ds(boff, SBLK)],
                                osrcl.at[pl.ds(v * CAP + w0 + boff, SBLK)])
                return 0

            lax.fori_loop(0, nw, wblk, 0)
            return 0

        lax.fori_loop(0, npass, ppass, 0)


_sort_bins = functools.partial(
    pl.kernel,
    out_type=[
        jax.ShapeDtypeStruct((NV * CAP,), jnp.int32),
        jax.ShapeDtypeStruct((NV * 160,), jnp.int32),
        jax.ShapeDtypeStruct((NV * 160,), jnp.float32),
        jax.ShapeDtypeStruct((NV * 16,), jnp.int32),
    ],
    mesh=_mesh,
    scratch_types=[
        pltpu.VMEM((SBLK,), jnp.int32),
        pltpu.VMEM((CAPS,), jnp.int32),
        pltpu.VMEM((176,), jnp.int32),
        pltpu.VMEM((176,), jnp.int32),
        pltpu.VMEM((176,), jnp.int32),
        pltpu.VMEM((32 * 176,), jnp.int32),
        pltpu.VMEM((48,), jnp.int32),
        pltpu.VMEM((160,), jnp.float32),
        pltpu.VMEM((16,), jnp.int32),
        pltpu.VMEM((48,), jnp.int32),
    ],
    compiler_params=_sc_params,
)(_sort_body)


def _agg_body(ncw, c, tbl, srcl, ooffl, cntl, *refs):
    (osum, omax, omin, srcb0, srcb1, idx0, idx1, gbuf0, gbuf1,
     accs, accm, accn, offb, cntb, sem0, sem1) = refs
    wid = _wid()

    zero = jnp.zeros((16,), jnp.float32)
    negs = zero + NEG
    poss = zero + POS
    NJ = W // 16
    bufs = ((srcb0, idx0, gbuf0, sem0), (srcb1, idx1, gbuf1, sem1))

    for b in range(NB):
        v = wid * NB + b

        def initr(i, _):
            sl = pl.ds(i * 16, 16)
            accs[sl] = zero
            accm[sl] = negs
            accn[sl] = poss
            return 0

        lax.fori_loop(0, ACC // 16, initr, 0)

        pltpu.sync_copy(cntl.at[pl.ds(v * 16, 16)], cntb)
        pltpu.sync_copy(ooffl.at[pl.ds(v * 160, 160)], offb.at[pl.ds(0, 160)])
        cnt = cntb[pl.ds(0, 16)][0]
        nblk = cnt // EBLK

        def issue(bi, p):
            srcb, idx2, gbuf, sem = bufs[p]
            boff = pl.multiple_of(bi * EBLK, 8)
            pltpu.sync_copy(srcl.at[pl.ds(v * CAP + boff, EBLK)], srcb)
            for i in range(EBLK // 16):
                vv = srcb[pl.ds(i * 16, 16)] * ncw + c
                idx2[i // 8, pl.ds((i % 8) * 16, 16)] = vv
            for j in range(EBLK // 128):
                pltpu.async_copy(tbl.at[idx2.at[j]],
                                 gbuf.at[pl.ds(j * 128, 128)], sem)

        def wait_g(p):
            srcb, idx2, gbuf, sem = bufs[p]
            for j in range(EBLK // 128):
                pltpu.make_async_copy(tbl.at[idx2.at[j]],
                                      gbuf.at[pl.ds(j * 128, 128)],
                                      sem).wait()

        def compute(bi, p, carry):
            gbuf = bufs[p][2]
            e0 = bi * EBLK

            def seg_cond(st):
                return st[1] < EBLK

            def seg_body(st):
                r = st[0]
                pos = st[1]
                regs = list(st[2:])
                rend = offb[pl.ds(r + 1, 16)][0] - e0
                send = jnp.minimum(rend, EBLK)

                def acc_e(el, regs2):
                    regs2 = list(regs2)
                    for j in range(NJ):
                        rr = gbuf[el, pl.ds(j * 16, 16)]
                        regs2[j] = regs2[j] + rr
                        regs2[NJ + j] = jnp.maximum(regs2[NJ + j], rr)
                        regs2[2 * NJ + j] = jnp.minimum(regs2[2 * NJ + j], rr)
                    return tuple(regs2)

                regs = list(lax.fori_loop(pos, send, acc_e, tuple(regs)))
                fin = rend <= EBLK

                def flush(args, rr=r):
                    for j in range(NJ):
                        sl = pl.ds(rr * W + j * 16, 16)
                        accs[sl] = accs[sl] + args[j]
                        accm[sl] = jnp.maximum(accm[sl], args[NJ + j])
                        accn[sl] = jnp.minimum(accn[sl], args[2 * NJ + j])
                    return ([zero] * NJ) + ([negs] * NJ) + ([poss] * NJ)

                regs = lax.cond(fin, flush, lambda a: list(a), tuple(regs))
                r = jnp.where(fin, r + 1, r)
                return (r, send, *regs)

            st = lax.while_loop(seg_cond, seg_body,
                                (carry[0], jnp.int32(0), *carry[1:]))
            return (st[0], *st[2:])

        issue(0, 0)

        def pair(i, carry):
            b0 = 2 * i
            b1 = 2 * i + 1
            wait_g(0)

            @pl.when(b1 < nblk)
            def _():
                issue(b1, 1)

            carry = compute(b0, 0, carry)

            def second(cc):
                @pl.when(b1 + 1 < nblk)
                def _():
                    issue(b1 + 1, 0)

                wait_g(1)
                return compute(b1, 1, cc)

            return lax.cond(b1 < nblk, second, lambda cc: cc, carry)

        init = (jnp.int32(0),) + tuple([zero] * NJ + [negs] * NJ + [poss] * NJ)
        lax.fori_loop(0, (nblk + 1) // 2, pair, init)

        pltpu.sync_copy(accs.at[pl.ds(0, R * W)],
                        osum.at[pl.ds(v * R * W, R * W)])
        pltpu.sync_copy(accm.at[pl.ds(0, R * W)],
                        omax.at[pl.ds(v * R * W, R * W)])
        pltpu.sync_copy(accn.at[pl.ds(0, R * W)],
                        omin.at[pl.ds(v * R * W, R * W)])


def _make_agg(ncw, c):
    outs = [jax.ShapeDtypeStruct((NPAD * W,), jnp.float32)] * 3
    scratch = [
        pltpu.VMEM((EBLK,), jnp.int32),
        pltpu.VMEM((EBLK,), jnp.int32),
        pltpu.VMEM((EBLK // 128, 128), jnp.int32),
        pltpu.VMEM((EBLK // 128, 128), jnp.int32),
        pltpu.VMEM((EBLK, W), jnp.float32),
        pltpu.VMEM((EBLK, W), jnp.float32),
        pltpu.VMEM((ACC,), jnp.float32),
        pltpu.VMEM((ACC,), jnp.float32),
        pltpu.VMEM((ACC,), jnp.float32),
        pltpu.VMEM((176,), jnp.int32),
        pltpu.VMEM((16,), jnp.int32),
        pltpu.SemaphoreType.DMA,
        pltpu.SemaphoreType.DMA,
    ]
    return pl.kernel(
        functools.partial(_agg_body, ncw, c),
        out_type=outs,
        mesh=_mesh,
        scratch_types=scratch,
        compiler_params=_sc_params,
    )


BR = 2000


def _c1_body(h_ref, sm_ref, mx_ref, mn_ref, dg_ref,
             wl0, wr0, wl1, wr1, wl2, wr2, bl0, bl1, bl2,
             opre, ostat):
    i = pl.program_id(0)
    deg = dg_ref[...]
    degc = jnp.maximum(deg, 1.0)
    emp = deg <= 0.0
    h = h_ref[...]
    mean = sm_ref[...] / degc
    mxv = jnp.where(emp, 0.0, mx_ref[...])
    mnv = jnp.where(emp, 0.0, mn_ref[...])
    parts = []
    for agg, Wl, bl, Wr in ((mean, wl0, bl0, wr0),
                            (mxv, wl1, bl1, wr1),
                            (mnv, wl2, bl2, wr2)):
        parts.append(
            jnp.dot(agg, Wl[...], preferred_element_type=jnp.float32)
            + bl[...]
            + jnp.dot(h, Wr[...], preferred_element_type=jnp.float32))
    pre = jnp.concatenate(parts, axis=1)
    opre[...] = pre

    @pl.when(i == 0)
    def _():
        ostat[...] = jnp.zeros_like(ostat)

    s0 = jnp.sum(pre, axis=0)[None, :]
    s1 = jnp.sum(pre * pre, axis=0)[None, :]
    pad = jnp.zeros((6, pre.shape[1]), jnp.float32)
    ostat[...] = ostat[...] + jnp.concatenate([s0, s1, pad], axis=0)


def _make_c1(K):
    grid = N // BR
    rb = lambda i: (i, 0)
    cb = lambda i: (0, 0)
    return pl.pallas_call(
        _c1_body,
        grid=(grid,),
        in_specs=[
            pl.BlockSpec((BR, K), rb),
            pl.BlockSpec((BR, K), rb),
            pl.BlockSpec((BR, K), rb),
            pl.BlockSpec((BR, K), rb),
            pl.BlockSpec((BR, 1), rb),
        ] + [pl.BlockSpec((K, HID), cb)] * 6 + [pl.BlockSpec((1, HID), cb)] * 3,
        out_specs=[
            pl.BlockSpec((BR, 3 * HID), rb),
            pl.BlockSpec((8, 3 * HID), cb),
        ],
        out_shape=[
            jax.ShapeDtypeStruct((N, 3 * HID), jnp.float32),
            jax.ShapeDtypeStruct((8, 3 * HID), jnp.float32),
        ],
    )


def _c2_body(final, pre_ref, stat_ref, g_ref, b_ref, *rest):
    if final:
        cw_ref, cb_ref, out_ref = rest
    else:
        (out_ref,) = rest
    stat = stat_ref[...]
    mu = stat[0:1, :] / jnp.float32(N)
    var = stat[1:2, :] / jnp.float32(N) - mu * mu
    inv = lax.rsqrt(var + 1e-5)
    h = (pre_ref[...] - mu) * (inv * g_ref[...]) + b_ref[...]
    h = jnp.maximum(h, 0.0)
    if final:
        out_ref[...] = (jnp.dot(h, cw_ref[...],
                                preferred_element_type=jnp.float32)
                        + cb_ref[...])
    else:
        out_ref[...] = h


def _make_c2(final):
    grid = N // BR
    rb = lambda i: (i, 0)
    cb = lambda i: (0, 0)
    K = 3 * HID
    in_specs = [
        pl.BlockSpec((BR, K), rb),
        pl.BlockSpec((8, K), cb),
        pl.BlockSpec((1, K), cb),
        pl.BlockSpec((1, K), cb),
    ]
    if final:
        in_specs += [pl.BlockSpec((K, HID), cb), pl.BlockSpec((1, HID), cb)]
        out_w = HID
    else:
        out_w = K
    return pl.pallas_call(
        functools.partial(_c2_body, final),
        grid=(grid,),
        in_specs=in_specs,
        out_specs=pl.BlockSpec((BR, out_w), rb),
        out_shape=jax.ShapeDtypeStruct((N, out_w), jnp.float32),
    )


def _layer_aggregate(tbl2d, ncw, srcl, ldstl, cnt):
    sums, maxs, mins = [], [], []
    for c in range(ncw):
        s, m, n = _make_agg(ncw, c)(tbl2d, srcl, ldstl, cnt)
        sums.append(s.reshape(NPAD, W))
        maxs.append(m.reshape(NPAD, W))
        mins.append(n.reshape(NPAD, W))
    sm = jnp.concatenate(sums, axis=1)[:N]
    mx = jnp.concatenate(maxs, axis=1)[:N]
    mn = jnp.concatenate(mins, axis=1)[:N]
    return sm, mx, mn


def kernel(x, edge_index,
           Wl_0_0, bl_0_0, Wr_0_0,
           Wl_0_1, bl_0_1, Wr_0_1,
           Wl_0_2, bl_0_2, Wr_0_2,
           bn_g_0, bn_b_0,
           Wl_1_0, bl_1_0, Wr_1_0,
           Wl_1_1, bl_1_1, Wr_1_1,
           Wl_1_2, bl_1_2, Wr_1_2,
           bn_g_1, bn_b_1,
           clf_W, clf_b):
    fragpk, fraghist = _part_edges(edge_index.reshape(2 * E))
    srcl, ldstl, deg, cnt = _sort_bins(fragpk, fraghist)

    sm0, mx0, mn0 = _layer_aggregate(x, D_IN // W, srcl, ldstl, cnt)
    degv = deg.reshape(NV, 160)[:, :R].reshape(NPAD, 1)[:N]
    c1 = _make_c1(D_IN)
    pre0, stat0 = c1(x, sm0, mx0, mn0, degv,
                     Wl_0_0, Wr_0_0, Wl_0_1, Wr_0_1, Wl_0_2, Wr_0_2,
                     bl_0_0.reshape(1, HID), bl_0_1.reshape(1, HID),
                     bl_0_2.reshape(1, HID))
    h1 = _make_c2(False)(pre0, stat0, bn_g_0.reshape(1, -1),
                         bn_b_0.reshape(1, -1))

    tbl1 = h1.reshape(N * (3 * HID // W), W)
    sm1, mx1, mn1 = _layer_aggregate(tbl1, 3 * HID // W, srcl, ldstl, cnt)
    c1b = _make_c1(3 * HID)
    pre1, stat1 = c1b(h1, sm1, mx1, mn1, degv,
                      Wl_1_0, Wr_1_0, Wl_1_1, Wr_1_1, Wl_1_2, Wr_1_2,
                      bl_1_0.reshape(1, HID), bl_1_1.reshape(1, HID),
                      bl_1_2.reshape(1, HID))
    clf_Wp = jnp.pad(clf_W, ((0, 0), (0, HID - clf_W.shape[1])))
    clf_bp = jnp.pad(clf_b, (0, HID - clf_b.shape[0])).reshape(1, HID)
    logits = _make_c2(True)(pre1, stat1, bn_g_1.reshape(1, -1),
                            bn_b_1.reshape(1, -1), clf_Wp, clf_bp)
    return logits[:, :clf_W.shape[1]]
